# Initial kernel scaffold; baseline (speedup 1.0000x reference)
import jax
import jax.numpy as jnp
from jax.experimental import pallas as pl


def kernel(keys, values, importance, W1, b1, W2, b2, T1, bt1, T2, bt2):
    raise NotImplementedError("write your pallas kernel here")


# speedup vs baseline: 4.0201x; 4.0201x over previous
import functools

import jax
import jax.numpy as jnp
import numpy as np
from jax.experimental import pallas as pl
from jax.experimental.pallas import tpu as pltpu

H = 1024
K = 32
KP = 128
THR = 0.1
TILE = 512
N = 4 * 4096


def _assign_accum_kernel(k_ref, v_ref, w1_ref, b1_ref, w2_ref, b2_ref,
                         idx_ref, cnt_ref, kc_ref, vc_ref):
    step = pl.program_id(0)

    k = k_ref[...]
    h = jax.lax.dot_general(k, w1_ref[...], (((1,), (0,)), ((), ())),
                            preferred_element_type=jnp.float32)
    h = jax.nn.relu(h + b1_ref[...])
    s = jax.lax.dot_general(h, w2_ref[...], (((1,), (0,)), ((), ())),
                            preferred_element_type=jnp.float32)
    s = s + b2_ref[...]
    idx = jnp.argmax(s, axis=-1, keepdims=True).astype(jnp.int32)
    idx_ref[...] = idx

    lane = jax.lax.broadcasted_iota(jnp.int32, (TILE, KP), 1)
    onehot = (lane == idx).astype(jnp.float32)

    cnt_p = jnp.sum(onehot, axis=0, keepdims=True)
    kc_p = jax.lax.dot_general(onehot, k, (((0,), (0,)), ((), ())),
                               preferred_element_type=jnp.float32)
    vc_p = jax.lax.dot_general(onehot, v_ref[...], (((0,), (0,)), ((), ())),
                               preferred_element_type=jnp.float32)

    @pl.when(step == 0)
    def _():
        cnt_ref[...] = jnp.zeros_like(cnt_ref)
        kc_ref[...] = jnp.zeros_like(kc_ref)
        vc_ref[...] = jnp.zeros_like(vc_ref)

    cnt_ref[...] += cnt_p
    kc_ref[...] += kc_p
    vc_ref[...] += vc_p


def _finalize_kernel(mix_ref, cc_ref, noise_ref, t1_ref, bt1_ref,
                     t2_ref, bt2_ref, out_ref):
    c = jax.lax.dot_general(mix_ref[...], cc_ref[...],
                            (((1,), (0,)), ((), ())),
                            preferred_element_type=jnp.float32)
    c = c + noise_ref[...]
    h = jax.lax.dot_general(c, t1_ref[...], (((1,), (0,)), ((), ())),
                            preferred_element_type=jnp.float32)
    h = jax.nn.relu(h + bt1_ref[...])
    o = jax.lax.dot_general(h, t2_ref[...], (((1,), (0,)), ((), ())),
                            preferred_element_type=jnp.float32)
    out_ref[...] = o + bt2_ref[...]


def _blend_kernel(k_ref, v_ref, imp_ref, idx_ref, tab_ref,
                  ok_ref, ov_ref):
    idx = idx_ref[...]
    lane = jax.lax.broadcasted_iota(jnp.int32, (TILE, 64), 1)
    oh_k = (lane == idx).astype(jnp.float32)
    oh_v = (lane == idx + 32).astype(jnp.float32)
    tab = tab_ref[...]
    gk = jax.lax.dot_general(oh_k, tab, (((1,), (0,)), ((), ())),
                             preferred_element_type=jnp.float32)
    gv = jax.lax.dot_general(oh_v, tab, (((1,), (0,)), ((), ())),
                             preferred_element_type=jnp.float32)
    m = imp_ref[...] > THR
    ok_ref[...] = jnp.where(m, k_ref[...], gk)
    ov_ref[...] = jnp.where(m, v_ref[...], gv)


def kernel(keys, values, importance, W1, b1, W2, b2, T1, bt1, T2, bt2):
    Bb, Ss, Hh = keys.shape
    n = Bb * Ss
    kf = keys.reshape(n, Hh)
    vf = values.reshape(n, Hh)
    imp = importance.reshape(n, 1)

    W2p = jnp.zeros((Hh // 2, KP), jnp.float32).at[:, :K].set(W2)
    b2p = jnp.full((1, KP), -1e30, jnp.float32).at[:, :K].set(b2)
    b1r = b1.reshape(1, Hh // 2)

    grid = n // TILE
    idx2, counts, kc, vc = pl.pallas_call(
        _assign_accum_kernel,
        grid=(grid,),
        in_specs=[
            pl.BlockSpec((TILE, Hh), lambda i: (i, 0)),
            pl.BlockSpec((TILE, Hh), lambda i: (i, 0)),
            pl.BlockSpec((Hh, Hh // 2), lambda i: (0, 0)),
            pl.BlockSpec((1, Hh // 2), lambda i: (0, 0)),
            pl.BlockSpec((Hh // 2, KP), lambda i: (0, 0)),
            pl.BlockSpec((1, KP), lambda i: (0, 0)),
        ],
        out_specs=[
            pl.BlockSpec((TILE, 1), lambda i: (i, 0)),
            pl.BlockSpec((1, KP), lambda i: (0, 0)),
            pl.BlockSpec((KP, Hh), lambda i: (0, 0)),
            pl.BlockSpec((KP, Hh), lambda i: (0, 0)),
        ],
        out_shape=[
            jax.ShapeDtypeStruct((n, 1), jnp.int32),
            jax.ShapeDtypeStruct((1, KP), jnp.float32),
            jax.ShapeDtypeStruct((KP, Hh), jnp.float32),
            jax.ShapeDtypeStruct((KP, Hh), jnp.float32),
        ],
    )(kf, vf, W1, b1r, W2p, b2p)

    cnt = counts[0, :K]
    nonempty = cnt > 0
    inv = jnp.where(nonempty, 1.0 / jnp.where(nonempty, cnt, 1.0), 1.0)
    _, top_idx = jax.lax.top_k(cnt, 3)
    rk = jax.random.key(1)
    sel = jax.random.randint(rk, (K,), 0, 3)
    src = top_idx[sel]
    eff_src = jnp.where(nonempty, jnp.arange(K), src)
    eff_scl = jnp.where(nonempty, inv, inv[src])
    mixA = (eff_scl[:, None]
            * (jnp.arange(K)[None, :] == eff_src[:, None]).astype(jnp.float32))
    mix = jnp.zeros((64, 64), jnp.float32)
    mix = mix.at[:K, :K].set(mixA).at[K:, K:].set(mixA)
    noise_k = jax.random.normal(jax.random.fold_in(rk, 1), (K, Hh)) * 0.1
    noise_v = jax.random.normal(jax.random.fold_in(rk, 2), (K, Hh)) * 0.1
    fill = (~nonempty)[:, None].astype(jnp.float32)
    noise = jnp.concatenate([noise_k * fill, noise_v * fill], axis=0)
    cc = jnp.concatenate([kc[:K], vc[:K]], axis=0)

    tab = pl.pallas_call(
        _finalize_kernel,
        out_shape=jax.ShapeDtypeStruct((64, Hh), jnp.float32),
    )(mix, cc, noise, T1, bt1.reshape(1, Hh), T2, bt2.reshape(1, Hh))

    out_k, out_v = pl.pallas_call(
        _blend_kernel,
        grid=(grid,),
        in_specs=[
            pl.BlockSpec((TILE, Hh), lambda i: (i, 0)),
            pl.BlockSpec((TILE, Hh), lambda i: (i, 0)),
            pl.BlockSpec((TILE, 1), lambda i: (i, 0)),
            pl.BlockSpec((TILE, 1), lambda i: (i, 0)),
            pl.BlockSpec((64, Hh), lambda i: (0, 0)),
        ],
        out_specs=[
            pl.BlockSpec((TILE, Hh), lambda i: (i, 0)),
            pl.BlockSpec((TILE, Hh), lambda i: (i, 0)),
        ],
        out_shape=[
            jax.ShapeDtypeStruct((n, Hh), jnp.float32),
            jax.ShapeDtypeStruct((n, Hh), jnp.float32),
        ],
    )(kf, vf, imp, idx2, tab)

    return (out_k.reshape(Bb, Ss, Hh), out_v.reshape(Bb, Ss, Hh))


# speedup vs baseline: 4.6425x; 1.1548x over previous
import jax
import jax.numpy as jnp
import numpy as np
from jax import lax
from jax.experimental import pallas as pl

H = 1024
K = 32
KP = 128
THR = 0.1
TILE = 512

_RK = jax.random.key(1)
_SEL64 = np.concatenate([np.asarray(jax.random.randint(_RK, (K,), 0, 3))] * 2
                        ).reshape(64, 1).astype(np.int32)
_NOISE64 = np.concatenate([
    np.asarray(jax.random.normal(jax.random.fold_in(_RK, 1), (K, H),
                                 dtype=jnp.float32) * 0.1),
    np.asarray(jax.random.normal(jax.random.fold_in(_RK, 2), (K, H),
                                 dtype=jnp.float32) * 0.1),
], axis=0)


def _assign_accum_kernel(k_ref, v_ref, w1_ref, b1_ref, w2_ref, b2_ref,
                         idx_ref, cnt_ref, kc_ref, vc_ref):
    step = pl.program_id(0)

    k = k_ref[...]
    h = jax.lax.dot_general(k, w1_ref[...], (((1,), (0,)), ((), ())),
                            preferred_element_type=jnp.float32)
    h = jax.nn.relu(h + b1_ref[...])
    s = jax.lax.dot_general(h, w2_ref[...], (((1,), (0,)), ((), ())),
                            preferred_element_type=jnp.float32)
    s = s + b2_ref[...]
    idx = jnp.argmax(s, axis=-1, keepdims=True).astype(jnp.int32)
    idx_ref[...] = idx

    lane = jax.lax.broadcasted_iota(jnp.int32, (TILE, KP), 1)
    onehot = (lane == idx).astype(jnp.float32)

    cnt_p = jnp.sum(onehot, axis=0, keepdims=True)
    kc_p = jax.lax.dot_general(onehot, k, (((0,), (0,)), ((), ())),
                               preferred_element_type=jnp.float32)
    vc_p = jax.lax.dot_general(onehot, v_ref[...], (((0,), (0,)), ((), ())),
                               preferred_element_type=jnp.float32)

    @pl.when(step == 0)
    def _():
        cnt_ref[...] = jnp.zeros_like(cnt_ref)
        kc_ref[...] = jnp.zeros_like(kc_ref)
        vc_ref[...] = jnp.zeros_like(vc_ref)

    cnt_ref[...] += cnt_p
    kc_ref[...] += kc_p
    vc_ref[...] += vc_p


def _finalize_kernel(cnt_ref, kc_ref, vc_ref, t1_ref, bt1_ref,
                     t2_ref, bt2_ref, noise_ref, sel_ref, out_ref):
    row = cnt_ref[...]
    lane = jax.lax.broadcasted_iota(jnp.int32, (1, KP), 1)
    m1 = jnp.max(row)
    i1 = jnp.argmax(row).astype(jnp.int32)
    row2 = jnp.where(lane == i1, -1.0, row)
    m2 = jnp.max(row2)
    i2 = jnp.argmax(row2).astype(jnp.int32)
    row3 = jnp.where(lane == i2, -1.0, row2)
    m3 = jnp.max(row3)
    i3 = jnp.argmax(row3).astype(jnp.int32)
    inv1 = jnp.where(m1 > 0, 1.0 / jnp.where(m1 > 0, m1, 1.0), 1.0)
    inv2 = jnp.where(m2 > 0, 1.0 / jnp.where(m2 > 0, m2, 1.0), 1.0)
    inv3 = jnp.where(m3 > 0, 1.0 / jnp.where(m3 > 0, m3, 1.0), 1.0)

    rc = jax.lax.broadcasted_iota(jnp.int32, (KP, KP), 0)
    cc_i = jax.lax.broadcasted_iota(jnp.int32, (KP, KP), 1)
    ident = (rc == cc_i).astype(jnp.float32)
    cnt_col = jax.lax.dot_general(ident, row, (((1,), (1,)), ((), ())),
                                  preferred_element_type=jnp.float32)
    cnt32 = jax.lax.slice(cnt_col, (0, 0), (K, 1))
    cnt64 = jnp.concatenate([cnt32, cnt32], axis=0)

    riota = jax.lax.broadcasted_iota(jnp.int32, (64, 1), 0)
    local = jnp.where(riota < K, riota, riota - K)
    offset = riota - local
    sel64 = sel_ref[...]
    src_l = jnp.where(sel64 == 0, i1, jnp.where(sel64 == 1, i2, i3))
    nonempty = cnt64 > 0.0
    eff_src = jnp.where(nonempty, local, src_l) + offset
    inv_own = 1.0 / jnp.maximum(cnt64, 1.0)
    scl_src = jnp.where(sel64 == 0, inv1,
                        jnp.where(sel64 == 1, inv2, inv3))
    eff_scl = jnp.where(nonempty, inv_own, scl_src)
    ciota = jax.lax.broadcasted_iota(jnp.int32, (64, 64), 1)
    mix = jnp.where(ciota == eff_src, eff_scl, 0.0)

    stack = jnp.concatenate([kc_ref[0:K, :], vc_ref[0:K, :]], axis=0)
    c = jax.lax.dot_general(mix, stack, (((1,), (0,)), ((), ())),
                            preferred_element_type=jnp.float32)
    c = c + noise_ref[...] * jnp.where(nonempty, 0.0, 1.0)
    hh = jax.lax.dot_general(c, t1_ref[...], (((1,), (0,)), ((), ())),
                             preferred_element_type=jnp.float32)
    hh = jax.nn.relu(hh + bt1_ref[...])
    o = jax.lax.dot_general(hh, t2_ref[...], (((1,), (0,)), ((), ())),
                            preferred_element_type=jnp.float32)
    out_ref[...] = o + bt2_ref[...]


def _blend_kernel(k_ref, v_ref, imp_ref, idx_ref, tab_ref,
                  ok_ref, ov_ref):
    idx = idx_ref[...]
    lane = jax.lax.broadcasted_iota(jnp.int32, (TILE, 64), 1)
    oh_k = (lane == idx).astype(jnp.float32)
    oh_v = (lane == idx + K).astype(jnp.float32)
    tab = tab_ref[...]
    gk = jax.lax.dot_general(oh_k, tab, (((1,), (0,)), ((), ())),
                             preferred_element_type=jnp.float32)
    gv = jax.lax.dot_general(oh_v, tab, (((1,), (0,)), ((), ())),
                             preferred_element_type=jnp.float32)
    m = imp_ref[...] > THR
    ok_ref[...] = jnp.where(m, k_ref[...], gk)
    ov_ref[...] = jnp.where(m, v_ref[...], gv)


def kernel(keys, values, importance, W1, b1, W2, b2, T1, bt1, T2, bt2):
    Bb, Ss, Hh = keys.shape
    n = Bb * Ss
    kf = keys.reshape(n, Hh)
    vf = values.reshape(n, Hh)
    imp = importance.reshape(n, 1)

    W2p = jnp.zeros((Hh // 2, KP), jnp.float32).at[:, :K].set(W2)
    b2p = jnp.full((1, KP), -1e30, jnp.float32).at[:, :K].set(b2)
    b1r = b1.reshape(1, Hh // 2)

    grid = n // TILE
    idx2, counts, kc, vc = pl.pallas_call(
        _assign_accum_kernel,
        grid=(grid,),
        in_specs=[
            pl.BlockSpec((TILE, Hh), lambda i: (i, 0)),
            pl.BlockSpec((TILE, Hh), lambda i: (i, 0)),
            pl.BlockSpec((Hh, Hh // 2), lambda i: (0, 0)),
            pl.BlockSpec((1, Hh // 2), lambda i: (0, 0)),
            pl.BlockSpec((Hh // 2, KP), lambda i: (0, 0)),
            pl.BlockSpec((1, KP), lambda i: (0, 0)),
        ],
        out_specs=[
            pl.BlockSpec((TILE, 1), lambda i: (i, 0)),
            pl.BlockSpec((1, KP), lambda i: (0, 0)),
            pl.BlockSpec((KP, Hh), lambda i: (0, 0)),
            pl.BlockSpec((KP, Hh), lambda i: (0, 0)),
        ],
        out_shape=[
            jax.ShapeDtypeStruct((n, 1), jnp.int32),
            jax.ShapeDtypeStruct((1, KP), jnp.float32),
            jax.ShapeDtypeStruct((KP, Hh), jnp.float32),
            jax.ShapeDtypeStruct((KP, Hh), jnp.float32),
        ],
    )(kf, vf, W1, b1r, W2p, b2p)

    tab = pl.pallas_call(
        _finalize_kernel,
        out_shape=jax.ShapeDtypeStruct((64, Hh), jnp.float32),
    )(counts, kc, vc, T1, bt1.reshape(1, Hh), T2, bt2.reshape(1, Hh),
      jnp.asarray(_NOISE64), jnp.asarray(_SEL64))

    out_k, out_v = pl.pallas_call(
        _blend_kernel,
        grid=(grid,),
        in_specs=[
            pl.BlockSpec((TILE, Hh), lambda i: (i, 0)),
            pl.BlockSpec((TILE, Hh), lambda i: (i, 0)),
            pl.BlockSpec((TILE, 1), lambda i: (i, 0)),
            pl.BlockSpec((TILE, 1), lambda i: (i, 0)),
            pl.BlockSpec((64, Hh), lambda i: (0, 0)),
        ],
        out_specs=[
            pl.BlockSpec((TILE, Hh), lambda i: (i, 0)),
            pl.BlockSpec((TILE, Hh), lambda i: (i, 0)),
        ],
        out_shape=[
            jax.ShapeDtypeStruct((n, Hh), jnp.float32),
            jax.ShapeDtypeStruct((n, Hh), jnp.float32),
        ],
    )(kf, vf, imp, idx2, tab)

    return (out_k.reshape(Bb, Ss, Hh), out_v.reshape(Bb, Ss, Hh))


# speedup vs baseline: 5.0104x; 1.0792x over previous
import jax
import jax.numpy as jnp
import numpy as np
from jax import lax
from jax.experimental import pallas as pl

H = 1024
K = 32
KP = 128
THR = 0.1
TILE = 1024

_RK = jax.random.key(1)
_SEL64 = np.concatenate([np.asarray(jax.random.randint(_RK, (K,), 0, 3))] * 2
                        ).reshape(64, 1).astype(np.int32)
_NOISE64 = np.concatenate([
    np.asarray(jax.random.normal(jax.random.fold_in(_RK, 1), (K, H),
                                 dtype=jnp.float32) * 0.1),
    np.asarray(jax.random.normal(jax.random.fold_in(_RK, 2), (K, H),
                                 dtype=jnp.float32) * 0.1),
], axis=0)


def _assign_accum_kernel(k_ref, v_ref, w1_ref, b1_ref, w2_ref, b2_ref,
                         idx_ref, cnt_ref, kc_ref, vc_ref):
    step = pl.program_id(0)

    k = k_ref[...]
    h = jax.lax.dot_general(k, w1_ref[...], (((1,), (0,)), ((), ())),
                            preferred_element_type=jnp.float32)
    h = jax.nn.relu(h + b1_ref[...])
    s = jax.lax.dot_general(h, w2_ref[...], (((1,), (0,)), ((), ())),
                            preferred_element_type=jnp.float32)
    s = s + b2_ref[...]
    idx = jnp.argmax(s, axis=-1, keepdims=True).astype(jnp.int32)
    idx_ref[...] = idx

    lane = jax.lax.broadcasted_iota(jnp.int32, (TILE, KP), 1)
    onehot = (lane == idx).astype(jnp.float32)

    cnt_p = jnp.sum(onehot, axis=0, keepdims=True)
    kc_p = jax.lax.dot_general(onehot, k, (((0,), (0,)), ((), ())),
                               preferred_element_type=jnp.float32)
    vc_p = jax.lax.dot_general(onehot, v_ref[...], (((0,), (0,)), ((), ())),
                               preferred_element_type=jnp.float32)

    @pl.when(step == 0)
    def _():
        cnt_ref[...] = jnp.zeros_like(cnt_ref)
        kc_ref[...] = jnp.zeros_like(kc_ref)
        vc_ref[...] = jnp.zeros_like(vc_ref)

    cnt_ref[...] += cnt_p
    kc_ref[...] += kc_p
    vc_ref[...] += vc_p


def _finalize_kernel(cnt_ref, kc_ref, vc_ref, t1_ref, bt1_ref,
                     t2_ref, bt2_ref, noise_ref, sel_ref, out_ref):
    row = cnt_ref[...]
    lane = jax.lax.broadcasted_iota(jnp.int32, (1, KP), 1)
    m1 = jnp.max(row)
    i1 = jnp.argmax(row).astype(jnp.int32)
    row2 = jnp.where(lane == i1, -1.0, row)
    m2 = jnp.max(row2)
    i2 = jnp.argmax(row2).astype(jnp.int32)
    row3 = jnp.where(lane == i2, -1.0, row2)
    m3 = jnp.max(row3)
    i3 = jnp.argmax(row3).astype(jnp.int32)
    inv1 = jnp.where(m1 > 0, 1.0 / jnp.where(m1 > 0, m1, 1.0), 1.0)
    inv2 = jnp.where(m2 > 0, 1.0 / jnp.where(m2 > 0, m2, 1.0), 1.0)
    inv3 = jnp.where(m3 > 0, 1.0 / jnp.where(m3 > 0, m3, 1.0), 1.0)

    rc = jax.lax.broadcasted_iota(jnp.int32, (KP, KP), 0)
    cc_i = jax.lax.broadcasted_iota(jnp.int32, (KP, KP), 1)
    ident = (rc == cc_i).astype(jnp.float32)
    cnt_col = jax.lax.dot_general(ident, row, (((1,), (1,)), ((), ())),
                                  preferred_element_type=jnp.float32)
    cnt32 = jax.lax.slice(cnt_col, (0, 0), (K, 1))
    cnt64 = jnp.concatenate([cnt32, cnt32], axis=0)

    riota = jax.lax.broadcasted_iota(jnp.int32, (64, 1), 0)
    local = jnp.where(riota < K, riota, riota - K)
    offset = riota - local
    sel64 = sel_ref[...]
    src_l = jnp.where(sel64 == 0, i1, jnp.where(sel64 == 1, i2, i3))
    nonempty = cnt64 > 0.0
    eff_src = jnp.where(nonempty, local, src_l) + offset
    inv_own = 1.0 / jnp.maximum(cnt64, 1.0)
    scl_src = jnp.where(sel64 == 0, inv1,
                        jnp.where(sel64 == 1, inv2, inv3))
    eff_scl = jnp.where(nonempty, inv_own, scl_src)
    ciota = jax.lax.broadcasted_iota(jnp.int32, (64, 64), 1)
    mix = jnp.where(ciota == eff_src, eff_scl, 0.0)

    stack = jnp.concatenate([kc_ref[0:K, :], vc_ref[0:K, :]], axis=0)
    c = jax.lax.dot_general(mix, stack, (((1,), (0,)), ((), ())),
                            preferred_element_type=jnp.float32)
    c = c + noise_ref[...] * jnp.where(nonempty, 0.0, 1.0)
    hh = jax.lax.dot_general(c, t1_ref[...], (((1,), (0,)), ((), ())),
                             preferred_element_type=jnp.float32)
    hh = jax.nn.relu(hh + bt1_ref[...])
    o = jax.lax.dot_general(hh, t2_ref[...], (((1,), (0,)), ((), ())),
                            preferred_element_type=jnp.float32)
    out_ref[...] = o + bt2_ref[...]


def _blend_kernel(k_ref, v_ref, imp_ref, idx_ref, tab_ref,
                  ok_ref, ov_ref):
    idx = idx_ref[...]
    lane = jax.lax.broadcasted_iota(jnp.int32, (TILE, 64), 1)
    oh_k = (lane == idx).astype(jnp.float32)
    oh_v = (lane == idx + K).astype(jnp.float32)
    tab = tab_ref[...]
    gk = jax.lax.dot_general(oh_k, tab, (((1,), (0,)), ((), ())),
                             preferred_element_type=jnp.float32)
    gv = jax.lax.dot_general(oh_v, tab, (((1,), (0,)), ((), ())),
                             preferred_element_type=jnp.float32)
    m = imp_ref[...] > THR
    ok_ref[...] = jnp.where(m, k_ref[...], gk)
    ov_ref[...] = jnp.where(m, v_ref[...], gv)


def kernel(keys, values, importance, W1, b1, W2, b2, T1, bt1, T2, bt2):
    Bb, Ss, Hh = keys.shape
    n = Bb * Ss
    kf = keys.reshape(n, Hh)
    vf = values.reshape(n, Hh)
    imp = importance.reshape(n, 1)

    W2p = jnp.zeros((Hh // 2, KP), jnp.float32).at[:, :K].set(W2)
    b2p = jnp.full((1, KP), -1e30, jnp.float32).at[:, :K].set(b2)
    b1r = b1.reshape(1, Hh // 2)

    grid = n // TILE
    idx2, counts, kc, vc = pl.pallas_call(
        _assign_accum_kernel,
        grid=(grid,),
        in_specs=[
            pl.BlockSpec((TILE, Hh), lambda i: (i, 0)),
            pl.BlockSpec((TILE, Hh), lambda i: (i, 0)),
            pl.BlockSpec((Hh, Hh // 2), lambda i: (0, 0)),
            pl.BlockSpec((1, Hh // 2), lambda i: (0, 0)),
            pl.BlockSpec((Hh // 2, KP), lambda i: (0, 0)),
            pl.BlockSpec((1, KP), lambda i: (0, 0)),
        ],
        out_specs=[
            pl.BlockSpec((TILE, 1), lambda i: (i, 0)),
            pl.BlockSpec((1, KP), lambda i: (0, 0)),
            pl.BlockSpec((KP, Hh), lambda i: (0, 0)),
            pl.BlockSpec((KP, Hh), lambda i: (0, 0)),
        ],
        out_shape=[
            jax.ShapeDtypeStruct((n, 1), jnp.int32),
            jax.ShapeDtypeStruct((1, KP), jnp.float32),
            jax.ShapeDtypeStruct((KP, Hh), jnp.float32),
            jax.ShapeDtypeStruct((KP, Hh), jnp.float32),
        ],
    )(kf, vf, W1, b1r, W2p, b2p)

    tab = pl.pallas_call(
        _finalize_kernel,
        out_shape=jax.ShapeDtypeStruct((64, Hh), jnp.float32),
    )(counts, kc, vc, T1, bt1.reshape(1, Hh), T2, bt2.reshape(1, Hh),
      jnp.asarray(_NOISE64), jnp.asarray(_SEL64))

    out_k, out_v = pl.pallas_call(
        _blend_kernel,
        grid=(grid,),
        in_specs=[
            pl.BlockSpec((TILE, Hh), lambda i: (i, 0)),
            pl.BlockSpec((TILE, Hh), lambda i: (i, 0)),
            pl.BlockSpec((TILE, 1), lambda i: (i, 0)),
            pl.BlockSpec((TILE, 1), lambda i: (i, 0)),
            pl.BlockSpec((64, Hh), lambda i: (0, 0)),
        ],
        out_specs=[
            pl.BlockSpec((TILE, Hh), lambda i: (i, 0)),
            pl.BlockSpec((TILE, Hh), lambda i: (i, 0)),
        ],
        out_shape=[
            jax.ShapeDtypeStruct((n, Hh), jnp.float32),
            jax.ShapeDtypeStruct((n, Hh), jnp.float32),
        ],
    )(kf, vf, imp, idx2, tab)

    return (out_k.reshape(Bb, Ss, Hh), out_v.reshape(Bb, Ss, Hh))


# speedup vs baseline: 5.0334x; 1.0046x over previous
import jax
import jax.numpy as jnp
import numpy as np
from jax import lax
from jax.experimental import pallas as pl
from jax.experimental.pallas import tpu as pltpu

H = 1024
K = 32
KP = 128
THR = 0.1
TILE = 1024

_RK = jax.random.key(1)
_SEL64 = np.concatenate([np.asarray(jax.random.randint(_RK, (K,), 0, 3))] * 2
                        ).reshape(64, 1).astype(np.int32)
_NOISE64 = np.concatenate([
    np.asarray(jax.random.normal(jax.random.fold_in(_RK, 1), (K, H),
                                 dtype=jnp.float32) * 0.1),
    np.asarray(jax.random.normal(jax.random.fold_in(_RK, 2), (K, H),
                                 dtype=jnp.float32) * 0.1),
], axis=0)


def _assign_accum_kernel(k_ref, v_ref, w1_ref, b1_ref, w2_ref, b2_ref,
                         idx_ref, cnt_ref, kc_ref, vc_ref):
    step = pl.program_id(0)

    k = k_ref[...]
    h = jax.lax.dot_general(k, w1_ref[...], (((1,), (0,)), ((), ())),
                            preferred_element_type=jnp.float32)
    h = jax.nn.relu(h + b1_ref[...])
    s = jax.lax.dot_general(h, w2_ref[...], (((1,), (0,)), ((), ())),
                            preferred_element_type=jnp.float32)
    s = s + b2_ref[...]
    idx = jnp.argmax(s, axis=-1, keepdims=True).astype(jnp.int32)
    idx_ref[...] = idx

    lane = jax.lax.broadcasted_iota(jnp.int32, (TILE, KP), 1)
    onehot = (lane == idx).astype(jnp.float32)

    cnt_p = jnp.sum(onehot, axis=0, keepdims=True)
    kc_p = jax.lax.dot_general(onehot, k, (((0,), (0,)), ((), ())),
                               preferred_element_type=jnp.float32)
    vc_p = jax.lax.dot_general(onehot, v_ref[...], (((0,), (0,)), ((), ())),
                               preferred_element_type=jnp.float32)

    @pl.when(step == 0)
    def _():
        cnt_ref[...] = jnp.zeros_like(cnt_ref)
        kc_ref[...] = jnp.zeros_like(kc_ref)
        vc_ref[...] = jnp.zeros_like(vc_ref)

    cnt_ref[...] += cnt_p
    kc_ref[...] += kc_p
    vc_ref[...] += vc_p


def _finalize_compute(cnt_ref, kc_ref, vc_ref, t1_ref, bt1_ref,
                      t2_ref, bt2_ref, noise_ref, sel_ref):
    row = cnt_ref[...]
    lane = jax.lax.broadcasted_iota(jnp.int32, (1, KP), 1)
    m1 = jnp.max(row)
    i1 = jnp.argmax(row).astype(jnp.int32)
    row2 = jnp.where(lane == i1, -1.0, row)
    m2 = jnp.max(row2)
    i2 = jnp.argmax(row2).astype(jnp.int32)
    row3 = jnp.where(lane == i2, -1.0, row2)
    m3 = jnp.max(row3)
    i3 = jnp.argmax(row3).astype(jnp.int32)
    inv1 = jnp.where(m1 > 0, 1.0 / jnp.where(m1 > 0, m1, 1.0), 1.0)
    inv2 = jnp.where(m2 > 0, 1.0 / jnp.where(m2 > 0, m2, 1.0), 1.0)
    inv3 = jnp.where(m3 > 0, 1.0 / jnp.where(m3 > 0, m3, 1.0), 1.0)

    rc = jax.lax.broadcasted_iota(jnp.int32, (KP, KP), 0)
    cc_i = jax.lax.broadcasted_iota(jnp.int32, (KP, KP), 1)
    ident = (rc == cc_i).astype(jnp.float32)
    cnt_col = jax.lax.dot_general(ident, row, (((1,), (1,)), ((), ())),
                                  preferred_element_type=jnp.float32)
    cnt32 = jax.lax.slice(cnt_col, (0, 0), (K, 1))
    cnt64 = jnp.concatenate([cnt32, cnt32], axis=0)

    riota = jax.lax.broadcasted_iota(jnp.int32, (64, 1), 0)
    local = jnp.where(riota < K, riota, riota - K)
    offset = riota - local
    sel64 = sel_ref[...]
    src_l = jnp.where(sel64 == 0, i1, jnp.where(sel64 == 1, i2, i3))
    nonempty = cnt64 > 0.0
    eff_src = jnp.where(nonempty, local, src_l) + offset
    inv_own = 1.0 / jnp.maximum(cnt64, 1.0)
    scl_src = jnp.where(sel64 == 0, inv1,
                        jnp.where(sel64 == 1, inv2, inv3))
    eff_scl = jnp.where(nonempty, inv_own, scl_src)
    ciota = jax.lax.broadcasted_iota(jnp.int32, (64, 64), 1)
    mix = jnp.where(ciota == eff_src, eff_scl, 0.0)

    stack = jnp.concatenate([kc_ref[0:K, :], vc_ref[0:K, :]], axis=0)
    c = jax.lax.dot_general(mix, stack, (((1,), (0,)), ((), ())),
                            preferred_element_type=jnp.float32)
    c = c + noise_ref[...] * jnp.where(nonempty, 0.0, 1.0)
    hh = jax.lax.dot_general(c, t1_ref[...], (((1,), (0,)), ((), ())),
                             preferred_element_type=jnp.float32)
    hh = jax.nn.relu(hh + bt1_ref[...])
    o = jax.lax.dot_general(hh, t2_ref[...], (((1,), (0,)), ((), ())),
                            preferred_element_type=jnp.float32)
    return o + bt2_ref[...]


def _blend_kernel(k_ref, v_ref, imp_ref, idx_ref, cnt_ref, kc_ref, vc_ref,
                  t1_ref, bt1_ref, t2_ref, bt2_ref, noise_ref, sel_ref,
                  ok_ref, ov_ref, tab_s):
    step = pl.program_id(0)

    @pl.when(step == 0)
    def _():
        tab_s[...] = _finalize_compute(cnt_ref, kc_ref, vc_ref, t1_ref,
                                       bt1_ref, t2_ref, bt2_ref,
                                       noise_ref, sel_ref)

    idx = idx_ref[...]
    lane = jax.lax.broadcasted_iota(jnp.int32, (TILE, 64), 1)
    oh_k = (lane == idx).astype(jnp.float32)
    oh_v = (lane == idx + K).astype(jnp.float32)
    tab = tab_s[...]
    gk = jax.lax.dot_general(oh_k, tab, (((1,), (0,)), ((), ())),
                             preferred_element_type=jnp.float32)
    gv = jax.lax.dot_general(oh_v, tab, (((1,), (0,)), ((), ())),
                             preferred_element_type=jnp.float32)
    m = imp_ref[...] > THR
    ok_ref[...] = jnp.where(m, k_ref[...], gk)
    ov_ref[...] = jnp.where(m, v_ref[...], gv)


def kernel(keys, values, importance, W1, b1, W2, b2, T1, bt1, T2, bt2):
    Bb, Ss, Hh = keys.shape
    n = Bb * Ss
    kf = keys.reshape(n, Hh)
    vf = values.reshape(n, Hh)
    imp = importance.reshape(n, 1)

    W2p = jnp.zeros((Hh // 2, KP), jnp.float32).at[:, :K].set(W2)
    b2p = jnp.full((1, KP), -1e30, jnp.float32).at[:, :K].set(b2)
    b1r = b1.reshape(1, Hh // 2)

    grid = n // TILE
    idx2, counts, kc, vc = pl.pallas_call(
        _assign_accum_kernel,
        grid=(grid,),
        in_specs=[
            pl.BlockSpec((TILE, Hh), lambda i: (i, 0)),
            pl.BlockSpec((TILE, Hh), lambda i: (i, 0)),
            pl.BlockSpec((Hh, Hh // 2), lambda i: (0, 0)),
            pl.BlockSpec((1, Hh // 2), lambda i: (0, 0)),
            pl.BlockSpec((Hh // 2, KP), lambda i: (0, 0)),
            pl.BlockSpec((1, KP), lambda i: (0, 0)),
        ],
        out_specs=[
            pl.BlockSpec((TILE, 1), lambda i: (i, 0)),
            pl.BlockSpec((1, KP), lambda i: (0, 0)),
            pl.BlockSpec((KP, Hh), lambda i: (0, 0)),
            pl.BlockSpec((KP, Hh), lambda i: (0, 0)),
        ],
        out_shape=[
            jax.ShapeDtypeStruct((n, 1), jnp.int32),
            jax.ShapeDtypeStruct((1, KP), jnp.float32),
            jax.ShapeDtypeStruct((KP, Hh), jnp.float32),
            jax.ShapeDtypeStruct((KP, Hh), jnp.float32),
        ],
    )(kf, vf, W1, b1r, W2p, b2p)

    out_k, out_v = pl.pallas_call(
        _blend_kernel,
        grid=(grid,),
        in_specs=[
            pl.BlockSpec((TILE, Hh), lambda i: (i, 0)),
            pl.BlockSpec((TILE, Hh), lambda i: (i, 0)),
            pl.BlockSpec((TILE, 1), lambda i: (i, 0)),
            pl.BlockSpec((TILE, 1), lambda i: (i, 0)),
            pl.BlockSpec((1, KP), lambda i: (0, 0)),
            pl.BlockSpec((KP, Hh), lambda i: (0, 0)),
            pl.BlockSpec((KP, Hh), lambda i: (0, 0)),
            pl.BlockSpec((Hh, Hh), lambda i: (0, 0)),
            pl.BlockSpec((1, Hh), lambda i: (0, 0)),
            pl.BlockSpec((Hh, Hh), lambda i: (0, 0)),
            pl.BlockSpec((1, Hh), lambda i: (0, 0)),
            pl.BlockSpec((64, Hh), lambda i: (0, 0)),
            pl.BlockSpec((64, 1), lambda i: (0, 0)),
        ],
        out_specs=[
            pl.BlockSpec((TILE, Hh), lambda i: (i, 0)),
            pl.BlockSpec((TILE, Hh), lambda i: (i, 0)),
        ],
        out_shape=[
            jax.ShapeDtypeStruct((n, Hh), jnp.float32),
            jax.ShapeDtypeStruct((n, Hh), jnp.float32),
        ],
        scratch_shapes=[pltpu.VMEM((64, Hh), jnp.float32)],
    )(kf, vf, imp, idx2, counts, kc, vc, T1, bt1.reshape(1, Hh),
      T2, bt2.reshape(1, Hh), jnp.asarray(_NOISE64), jnp.asarray(_SEL64))

    return (out_k.reshape(Bb, Ss, Hh), out_v.reshape(Bb, Ss, Hh))


# speedup vs baseline: 5.1315x; 1.0195x over previous
import jax
import jax.numpy as jnp
import numpy as np
from jax import lax
from jax.experimental import pallas as pl
from jax.experimental.pallas import tpu as pltpu

H = 1024
K = 32
KP = 128
THR = 0.1
TILE = 1024

_RK = jax.random.key(1)
_SEL64 = np.concatenate([np.asarray(jax.random.randint(_RK, (K,), 0, 3))] * 2
                        ).reshape(64, 1).astype(np.int32)
_NOISE64 = np.concatenate([
    np.asarray(jax.random.normal(jax.random.fold_in(_RK, 1), (K, H),
                                 dtype=jnp.float32) * 0.1),
    np.asarray(jax.random.normal(jax.random.fold_in(_RK, 2), (K, H),
                                 dtype=jnp.float32) * 0.1),
], axis=0)


def _assign_accum_kernel(k_ref, v_ref, w1_ref, b1_ref, w2_ref, b2_ref,
                         idx_ref, cnt_ref, kc_ref, vc_ref):
    step = pl.program_id(0)

    k = k_ref[...]
    h = jax.lax.dot_general(k, w1_ref[...], (((1,), (0,)), ((), ())),
                            preferred_element_type=jnp.float32)
    h = jax.nn.relu(h + b1_ref[...])
    s = jax.lax.dot_general(h, w2_ref[...], (((1,), (0,)), ((), ())),
                            preferred_element_type=jnp.float32)
    s = s + b2_ref[...]
    idx = jnp.argmax(s, axis=-1, keepdims=True).astype(jnp.int32)
    idx_ref[...] = idx

    lane = jax.lax.broadcasted_iota(jnp.int32, (TILE, KP), 1)
    onehot = (lane == idx).astype(jnp.float32)

    cnt_p = jnp.sum(onehot, axis=0, keepdims=True)
    kc_p = jax.lax.dot_general(onehot, k, (((0,), (0,)), ((), ())),
                               preferred_element_type=jnp.float32)
    vc_p = jax.lax.dot_general(onehot, v_ref[...], (((0,), (0,)), ((), ())),
                               preferred_element_type=jnp.float32)

    @pl.when(step == 0)
    def _():
        cnt_ref[...] = jnp.zeros_like(cnt_ref)
        kc_ref[...] = jnp.zeros_like(kc_ref)
        vc_ref[...] = jnp.zeros_like(vc_ref)

    cnt_ref[...] += cnt_p
    kc_ref[...] += kc_p
    vc_ref[...] += vc_p


def _finalize_compute(cnt_ref, kc_ref, vc_ref, t1_ref, bt1_ref,
                      t2_ref, bt2_ref, noise_ref, sel_ref):
    row = cnt_ref[...]
    lane = jax.lax.broadcasted_iota(jnp.int32, (1, KP), 1)
    m1 = jnp.max(row)
    i1 = jnp.argmax(row).astype(jnp.int32)
    row2 = jnp.where(lane == i1, -1.0, row)
    m2 = jnp.max(row2)
    i2 = jnp.argmax(row2).astype(jnp.int32)
    row3 = jnp.where(lane == i2, -1.0, row2)
    m3 = jnp.max(row3)
    i3 = jnp.argmax(row3).astype(jnp.int32)
    inv1 = jnp.where(m1 > 0, 1.0 / jnp.where(m1 > 0, m1, 1.0), 1.0)
    inv2 = jnp.where(m2 > 0, 1.0 / jnp.where(m2 > 0, m2, 1.0), 1.0)
    inv3 = jnp.where(m3 > 0, 1.0 / jnp.where(m3 > 0, m3, 1.0), 1.0)

    rc = jax.lax.broadcasted_iota(jnp.int32, (KP, KP), 0)
    cc_i = jax.lax.broadcasted_iota(jnp.int32, (KP, KP), 1)
    ident = (rc == cc_i).astype(jnp.float32)
    cnt_col = jax.lax.dot_general(ident, row, (((1,), (1,)), ((), ())),
                                  preferred_element_type=jnp.float32)
    cnt32 = jax.lax.slice(cnt_col, (0, 0), (K, 1))
    cnt64 = jnp.concatenate([cnt32, cnt32], axis=0)

    riota = jax.lax.broadcasted_iota(jnp.int32, (64, 1), 0)
    local = jnp.where(riota < K, riota, riota - K)
    offset = riota - local
    sel64 = sel_ref[...]
    src_l = jnp.where(sel64 == 0, i1, jnp.where(sel64 == 1, i2, i3))
    nonempty = cnt64 > 0.0
    eff_src = jnp.where(nonempty, local, src_l) + offset
    inv_own = 1.0 / jnp.maximum(cnt64, 1.0)
    scl_src = jnp.where(sel64 == 0, inv1,
                        jnp.where(sel64 == 1, inv2, inv3))
    eff_scl = jnp.where(nonempty, inv_own, scl_src)
    ciota = jax.lax.broadcasted_iota(jnp.int32, (64, 64), 1)
    mix = jnp.where(ciota == eff_src, eff_scl, 0.0)

    stack = jnp.concatenate([kc_ref[0:K, :], vc_ref[0:K, :]], axis=0)
    c = jax.lax.dot_general(mix, stack, (((1,), (0,)), ((), ())),
                            preferred_element_type=jnp.float32)
    c = c + noise_ref[...] * jnp.where(nonempty, 0.0, 1.0)
    hh = jax.lax.dot_general(c, t1_ref[...], (((1,), (0,)), ((), ())),
                             preferred_element_type=jnp.float32)
    hh = jax.nn.relu(hh + bt1_ref[...])
    o = jax.lax.dot_general(hh, t2_ref[...], (((1,), (0,)), ((), ())),
                            preferred_element_type=jnp.float32)
    return o + bt2_ref[...]


def _blend_kernel(k_ref, v_ref, imp_ref, idx_ref, cnt_ref, kc_ref, vc_ref,
                  t1_ref, bt1_ref, t2_ref, bt2_ref, noise_ref, sel_ref,
                  ok_ref, ov_ref, tab_s):
    step = pl.program_id(0)

    @pl.when(step == 0)
    def _():
        tab_s[...] = _finalize_compute(cnt_ref, kc_ref, vc_ref, t1_ref,
                                       bt1_ref, t2_ref, bt2_ref,
                                       noise_ref, sel_ref)

    idx = idx_ref[...]
    lane = jax.lax.broadcasted_iota(jnp.int32, (TILE, 64), 1)
    oh_k = (lane == idx).astype(jnp.float32)
    oh_v = (lane == idx + K).astype(jnp.float32)
    tab = tab_s[...]
    gk = jax.lax.dot_general(oh_k, tab, (((1,), (0,)), ((), ())),
                             preferred_element_type=jnp.float32)
    gv = jax.lax.dot_general(oh_v, tab, (((1,), (0,)), ((), ())),
                             preferred_element_type=jnp.float32)
    m = imp_ref[...] > THR
    ok_ref[...] = jnp.where(m, k_ref[...], gk)
    ov_ref[...] = jnp.where(m, v_ref[...], gv)


def kernel(keys, values, importance, W1, b1, W2, b2, T1, bt1, T2, bt2):
    Bb, Ss, Hh = keys.shape
    n = Bb * Ss
    kf = keys.reshape(n, Hh)
    vf = values.reshape(n, Hh)
    imp = importance.reshape(n, 1)

    b1r = b1.reshape(1, Hh // 2)

    grid = n // TILE
    idx2, counts, kc, vc = pl.pallas_call(
        _assign_accum_kernel,
        grid=(grid,),
        in_specs=[
            pl.BlockSpec((TILE, Hh), lambda i: (i, 0)),
            pl.BlockSpec((TILE, Hh), lambda i: (i, 0)),
            pl.BlockSpec((Hh, Hh // 2), lambda i: (0, 0)),
            pl.BlockSpec((1, Hh // 2), lambda i: (0, 0)),
            pl.BlockSpec((Hh // 2, K), lambda i: (0, 0)),
            pl.BlockSpec((1, K), lambda i: (0, 0)),
        ],
        out_specs=[
            pl.BlockSpec((TILE, 1), lambda i: (i, 0)),
            pl.BlockSpec((1, KP), lambda i: (0, 0)),
            pl.BlockSpec((KP, Hh), lambda i: (0, 0)),
            pl.BlockSpec((KP, Hh), lambda i: (0, 0)),
        ],
        out_shape=[
            jax.ShapeDtypeStruct((n, 1), jnp.int32),
            jax.ShapeDtypeStruct((1, KP), jnp.float32),
            jax.ShapeDtypeStruct((KP, Hh), jnp.float32),
            jax.ShapeDtypeStruct((KP, Hh), jnp.float32),
        ],
    )(kf, vf, W1, b1r, W2, b2.reshape(1, K))

    out_k, out_v = pl.pallas_call(
        _blend_kernel,
        grid=(grid,),
        in_specs=[
            pl.BlockSpec((TILE, Hh), lambda i: (i, 0)),
            pl.BlockSpec((TILE, Hh), lambda i: (i, 0)),
            pl.BlockSpec((TILE, 1), lambda i: (i, 0)),
            pl.BlockSpec((TILE, 1), lambda i: (i, 0)),
            pl.BlockSpec((1, KP), lambda i: (0, 0)),
            pl.BlockSpec((KP, Hh), lambda i: (0, 0)),
            pl.BlockSpec((KP, Hh), lambda i: (0, 0)),
            pl.BlockSpec((Hh, Hh), lambda i: (0, 0)),
            pl.BlockSpec((1, Hh), lambda i: (0, 0)),
            pl.BlockSpec((Hh, Hh), lambda i: (0, 0)),
            pl.BlockSpec((1, Hh), lambda i: (0, 0)),
            pl.BlockSpec((64, Hh), lambda i: (0, 0)),
            pl.BlockSpec((64, 1), lambda i: (0, 0)),
        ],
        out_specs=[
            pl.BlockSpec((TILE, Hh), lambda i: (i, 0)),
            pl.BlockSpec((TILE, Hh), lambda i: (i, 0)),
        ],
        out_shape=[
            jax.ShapeDtypeStruct((n, Hh), jnp.float32),
            jax.ShapeDtypeStruct((n, Hh), jnp.float32),
        ],
        scratch_shapes=[pltpu.VMEM((64, Hh), jnp.float32)],
    )(kf, vf, imp, idx2, counts, kc, vc, T1, bt1.reshape(1, Hh),
      T2, bt2.reshape(1, Hh), jnp.asarray(_NOISE64), jnp.asarray(_SEL64))

    return (out_k.reshape(Bb, Ss, Hh), out_v.reshape(Bb, Ss, Hh))
